# baseline (speedup 1.0000x reference)
import functools

import jax
import jax.numpy as jnp
from jax import lax
from jax.experimental import pallas as pl
from jax.experimental.pallas import tpu as pltpu
from jax.experimental.pallas import tpu_sc as plsc

N_NODES = 10000
N_FEATURES = 128
N_TEMPLATES = 16
N_TNODES = 16
N_CLASSES = 8

NC = 2
NS = 16
NW = NC * NS
CHUNK = 128
NP = 10112
ROWS_PER_TILE = NP // NS


def _dense_body(x_ref, tf_ref, lt_ref, w_ref, b_ref, pext_ref, base_ref):
    T, n, F = N_TEMPLATES, N_TNODES, N_FEATURES
    tf2 = tf_ref[...].reshape(T * n, F)
    w = w_ref[...]
    wex = jnp.broadcast_to(w[:, None, :], (T, n, N_CLASSES)).reshape(
        T * n, N_CLASSES) * (1.0 / n)
    fbw = lax.dot_general(tf2, wex, (((0,), (0,)), ((), ())))
    f2 = jnp.sum(tf2 * tf2, axis=1, keepdims=True)
    f2w = lax.dot_general(f2, wex, (((0,), (0,)), ((), ())))
    sw = jnp.sum(w, axis=0, keepdims=True)
    struct = jnp.mean(lt_ref[...].reshape(T, n * n), axis=1, keepdims=True)
    c0 = jnp.sum(struct * w, axis=0, keepdims=True) + b_ref[...]

    x = x_ref[...]
    x2 = jnp.sum(x * x, axis=1, keepdims=True)
    p = x2 * sw + f2w - 2.0 * jnp.dot(x, fbw)
    half_p = 0.5 * p
    base_ref[...] = half_p + c0
    rows = jnp.concatenate(
        [half_p,
         jnp.ones((N_NODES, 1), jnp.float32),
         jnp.zeros((N_NODES, 16 - N_CLASSES - 1), jnp.float32)], axis=1)
    pext_ref[...] = jnp.concatenate(
        [rows, jnp.zeros((NP - N_NODES, 16), jnp.float32)], axis=0)


def _dense_part(x, tf, lt, w, b2):
    return pl.pallas_call(
        _dense_body,
        out_shape=(
            jax.ShapeDtypeStruct((NP, 16), jnp.float32),
            jax.ShapeDtypeStruct((N_NODES, N_CLASSES), jnp.float32),
        ),
    )(x, tf, lt, w, b2)


def _make_sc_kernel(k_chunks):
    mesh = plsc.VectorSubcoreMesh(core_axis_name="c", subcore_axis_name="s")

    @functools.partial(
        pl.kernel,
        out_type=jax.ShapeDtypeStruct((NC, NP, 16), jnp.float32),
        mesh=mesh,
        scratch_types=[
            pltpu.VMEM((k_chunks, CHUNK), jnp.int32),
            pltpu.VMEM((k_chunks, CHUNK), jnp.int32),
            pltpu.VMEM((8, CHUNK, 16), jnp.float32),
            pltpu.VMEM((ROWS_PER_TILE, 16), jnp.float32),
            pltpu.VMEM_SHARED((NP, 16), jnp.float32),
            pltpu.VMEM_SHARED((NP, 16), jnp.float32),
        ] + [pltpu.SemaphoreType.DMA] * 16,
        compiler_params=pltpu.CompilerParams(use_tc_tiling_on_sc=False),
    )
    def scatter_kernel(src_hbm, dst_hbm, pext_hbm, out_hbm,
                       idx_s, idx_d, bufring, stage, acc, ptab, *sems):
        c = lax.axis_index("c")
        s = lax.axis_index("s")
        wid = s * NC + c
        row0 = s * ROWS_PER_TILE

        pltpu.sync_copy(src_hbm.at[wid], idx_s)
        pltpu.sync_copy(dst_hbm.at[wid], idx_d)
        pltpu.sync_copy(pext_hbm.at[pl.ds(row0, ROWS_PER_TILE)],
                        ptab.at[pl.ds(row0, ROWS_PER_TILE)])

        def _zero_row(i, carry):
            stage[i, :] = jnp.zeros((16,), jnp.float32)
            return carry
        lax.fori_loop(0, ROWS_PER_TILE, _zero_row, None)
        pltpu.sync_copy(stage, acc.at[pl.ds(row0, ROWS_PER_TILE)])
        plsc.subcore_barrier()

        nbuf = 8
        gsems = sems[:nbuf]
        ssems = sems[nbuf:]

        def _gstart(j, slot):
            pltpu.async_copy(ptab.at[idx_s.at[j]], bufring.at[slot],
                             gsems[slot])

        def _gwait(j, slot):
            pltpu.make_async_copy(ptab.at[idx_s.at[j]], bufring.at[slot],
                                  gsems[slot]).wait()

        def _sstart(j, slot):
            pltpu.async_copy(bufring.at[slot], acc.at[idx_d.at[j]],
                             ssems[slot], add=True)

        def _swait(j, slot):
            pltpu.make_async_copy(bufring.at[slot], acc.at[idx_d.at[j]],
                                  ssems[slot]).wait()

        _gstart(0, 0)
        _gstart(1, 1)

        def _pipe(i, carry):
            j0 = nbuf * i
            for t in range(nbuf):
                j = j0 + t
                nslot = (t + 2) % nbuf

                @pl.when(j >= 6)
                def _():
                    _swait(j - 6, nslot)

                @pl.when(j + 2 < k_chunks)
                def _():
                    _gstart(j + 2, nslot)
                _gwait(j, t)
                _sstart(j, t)
            return carry
        lax.fori_loop(0, k_chunks // nbuf, _pipe, None)
        for t in range(6):
            j = k_chunks - 6 + t
            _swait(j, j % nbuf)
        plsc.subcore_barrier()

        pltpu.sync_copy(acc.at[pl.ds(row0, ROWS_PER_TILE)],
                        out_hbm.at[c, pl.ds(row0, ROWS_PER_TILE)])

    return scatter_kernel


def _combine_body(base_ref, ap_ref, out_ref):
    a = ap_ref[0] + ap_ref[1]
    ssum = a[:N_NODES, :N_CLASSES]
    deg = a[:N_NODES, N_CLASSES:N_CLASSES + 1]
    out_ref[...] = base_ref[...] + ssum / jnp.maximum(deg, 1.0)


def _combine(base, apart):
    return pl.pallas_call(
        _combine_body,
        out_shape=jax.ShapeDtypeStruct((N_NODES, N_CLASSES), jnp.float32),
    )(base, apart)


def kernel(x, edge_index, latent_template, templates_features, W, b):
    e = edge_index.shape[1]
    k_chunks = -(-e // (NW * CHUNK))
    k_chunks = -(-k_chunks // 8) * 8
    per_w = k_chunks * CHUNK
    e_pad = per_w * NW

    pad = jnp.full((e_pad - e,), N_NODES, jnp.int32)
    src = jnp.concatenate([edge_index[0], pad]).reshape(NW, k_chunks, CHUNK)
    dst = jnp.concatenate([edge_index[1], pad]).reshape(NW, k_chunks, CHUNK)

    pext, base = _dense_part(x, templates_features, latent_template,
                             W, b.reshape(1, N_CLASSES))
    apart = _make_sc_kernel(k_chunks)(src, dst, pext)
    return _combine(base, apart)
